# baseline (speedup 1.0000x reference)
import functools

import jax
import jax.numpy as jnp
from jax import lax
from jax.experimental import pallas as pl
from jax.experimental.pallas import tpu as pltpu
from jax.experimental.pallas import tpu_sc as plsc

B = 16384
N_FIELDS = 26
VOCAB = 100000
EMBED_DIM = 64
N_DENSE = 13
H1 = 1024
H2 = 512

ROWS = B * N_FIELDS
NW = 32
RPW = ROWS // NW
CHUNK = 128
NCH = RPW // CHUNK
VPC = CHUNK // 16


def _gather_body(xv_hbm, table_hbm, out_hbm, idx_v, rows_v, sem):
    wid = lax.axis_index("c") * 16 + lax.axis_index("s")
    base = wid * RPW

    pltpu.sync_copy(xv_hbm.at[pl.ds(wid * NCH, NCH)], idx_v)

    lanes = lax.iota(jnp.int32, 16)

    def off_chunk(c, _):
        def off_vreg(k, _):
            pos = base + c * CHUNK + k * 16 + lanes
            f = lax.rem(pos, N_FIELDS)
            idx_v[c, pl.ds(k * 16, 16)] = idx_v[c, pl.ds(k * 16, 16)] + f * VOCAB
            return 0
        return lax.fori_loop(0, VPC, off_vreg, 0)

    lax.fori_loop(0, NCH, off_chunk, 0)

    def gather_chunk(c, _):
        pltpu.async_copy(table_hbm.at[idx_v.at[c]], rows_v, sem).wait()
        pltpu.sync_copy(rows_v, out_hbm.at[pl.ds(base + c * CHUNK, CHUNK)])
        return 0

    lax.fori_loop(0, NCH, gather_chunk, 0)


_gather = functools.partial(
    pl.kernel,
    mesh=plsc.VectorSubcoreMesh(core_axis_name="c", subcore_axis_name="s"),
    compiler_params=pltpu.CompilerParams(use_tc_tiling_on_sc=False),
    out_type=jax.ShapeDtypeStruct((ROWS, EMBED_DIM), jnp.float32),
    scratch_types=[
        pltpu.VMEM((NCH, CHUNK), jnp.int32),
        pltpu.VMEM((CHUNK, EMBED_DIM), jnp.float32),
        pltpu.SemaphoreType.DMA,
    ],
)(_gather_body)


BB = 512


def _mlp_body(xi_ref, emb_ref, w1d_ref, w1e_ref, b1_ref, w2_ref, b2_ref,
              w3_ref, b3_ref, o_ref):
    h1 = jnp.dot(emb_ref[...], w1e_ref[...], preferred_element_type=jnp.float32)
    h1 = h1 + jnp.dot(xi_ref[...], w1d_ref[...], preferred_element_type=jnp.float32)
    h1 = jnp.maximum(h1 + b1_ref[...], 0.0)
    h2 = jnp.dot(h1, w2_ref[...], preferred_element_type=jnp.float32)
    h2 = jnp.maximum(h2 + b2_ref[...], 0.0)
    y = jnp.dot(h2, w3_ref[...], preferred_element_type=jnp.float32) + b3_ref[...]
    o_ref[...] = jax.nn.sigmoid(y)


def kernel(xi, xv, emb_tables, W1, b1, W2, b2, W3, b3):
    table_flat = emb_tables.reshape(N_FIELDS * VOCAB, EMBED_DIM)
    xv_flat = xv.reshape(ROWS // CHUNK, CHUNK).astype(jnp.int32)

    gathered = _gather(xv_flat, table_flat)
    emb_flat = gathered.reshape(B, N_FIELDS * EMBED_DIM)

    W1d = W1[:N_DENSE]
    W1e = W1[N_DENSE:]
    EIN = N_FIELDS * EMBED_DIM

    out = pl.pallas_call(
        _mlp_body,
        grid=(B // BB,),
        in_specs=[
            pl.BlockSpec((BB, N_DENSE), lambda i: (i, 0)),
            pl.BlockSpec((BB, EIN), lambda i: (i, 0)),
            pl.BlockSpec((N_DENSE, H1), lambda i: (0, 0)),
            pl.BlockSpec((EIN, H1), lambda i: (0, 0)),
            pl.BlockSpec((1, H1), lambda i: (0, 0)),
            pl.BlockSpec((H1, H2), lambda i: (0, 0)),
            pl.BlockSpec((1, H2), lambda i: (0, 0)),
            pl.BlockSpec((H2, 1), lambda i: (0, 0)),
            pl.BlockSpec((1, 1), lambda i: (0, 0)),
        ],
        out_specs=pl.BlockSpec((BB, 1), lambda i: (i, 0)),
        out_shape=jax.ShapeDtypeStruct((B, 1), jnp.float32),
    )(xi, emb_flat, W1d, W1e, b1.reshape(1, H1), W2, b2.reshape(1, H2),
      W3, b3.reshape(1, 1))
    return out
